# Initial kernel scaffold; baseline (speedup 1.0000x reference)
import jax
import jax.numpy as jnp
from jax.experimental import pallas as pl


def kernel(x, edge_index, W1, b1, W2, b2):
    raise NotImplementedError("write your pallas kernel here")


# speedup vs baseline: 10.2703x; 10.2703x over previous
import functools

import jax
import jax.numpy as jnp
from jax import lax
from jax.experimental import pallas as pl
from jax.experimental.pallas import tpu as pltpu
from jax.experimental.pallas import tpu_sc as plsc

N = 10000
E = 160000
D_IN = 296
D_HID = 256
H = 128
NC = 2
NS = 16
NPAD = 10240
SL = NPAD // NS
EPT_DEG = E // (NC * NS)
EPT_AGG = E // NS
CHUNK = 128
NFULL = EPT_AGG // CHUNK
REM = EPT_AGG - NFULL * CHUNK
R = 2000
NBLK = N // R


def _deg_body(dst_hbm, degp_hbm, dst_v, deg_v, tmp_v, slots):
    c = lax.axis_index("c")
    s = lax.axis_index("s")
    zeros16 = jnp.zeros((16,), jnp.float32)
    ones16 = jnp.ones((16,), jnp.float32)
    lane = jax.lax.iota(jnp.int32, 16)

    def zloop(i, carry):
        deg_v[pl.ds(i * 16, 16)] = zeros16
        return carry
    lax.fori_loop(0, NPAD // 16, zloop, None)

    base = (c * NS + s) * EPT_DEG
    pltpu.sync_copy(dst_hbm.at[pl.ds(base, EPT_DEG)], dst_v.at[pl.ds(0, EPT_DEG)])

    def hloop(i, carry):
        idx = dst_v[pl.ds(i * 16, 16)]
        plsc.addupdate_scatter(deg_v, [idx], ones16)
        return carry
    nfull = EPT_DEG // 16
    lax.fori_loop(0, nfull, hloop, None)
    rem_mask = lane < (EPT_DEG - nfull * 16)
    idx = dst_v[pl.ds(nfull * 16, 16)]
    idx = jnp.where(rem_mask, idx, 0)
    plsc.addupdate_scatter(deg_v, [idx], ones16, mask=rem_mask)

    pltpu.sync_copy(deg_v, slots.at[s])
    plsc.subcore_barrier()
    for j in range(NS):
        pltpu.sync_copy(slots.at[j, pl.ds(s * SL, SL)], tmp_v)
        if j == 0:
            def cploop(k, carry):
                deg_v[pl.ds(k * 16, 16)] = tmp_v[pl.ds(k * 16, 16)]
                return carry
            lax.fori_loop(0, SL // 16, cploop, None)
        else:
            def adloop(k, carry):
                deg_v[pl.ds(k * 16, 16)] = (deg_v[pl.ds(k * 16, 16)]
                                            + tmp_v[pl.ds(k * 16, 16)])
                return carry
            lax.fori_loop(0, SL // 16, adloop, None)
    pltpu.sync_copy(deg_v.at[pl.ds(0, SL)],
                    degp_hbm.at[pl.ds(c * NPAD + s * SL, SL)])


@functools.cache
def _sc_calls():
    mesh = plsc.VectorSubcoreMesh(core_axis_name="c", subcore_axis_name="s",
                                  num_cores=NC, num_subcores=NS)
    deg_call = pl.kernel(
        _deg_body,
        out_type=jax.ShapeDtypeStruct((NC * NPAD,), jnp.float32),
        mesh=mesh,
        compiler_params=pltpu.CompilerParams(needs_layout_passes=False),
        scratch_types=[
            pltpu.VMEM((EPT_DEG + 16,), jnp.int32),
            pltpu.VMEM((NPAD,), jnp.float32),
            pltpu.VMEM((SL,), jnp.float32),
            pltpu.VMEM_SHARED((NS, NPAD), jnp.float32),
        ],
    )
    agg_call = pl.kernel(
        _agg_body,
        out_type=jax.ShapeDtypeStruct((NC * N, H), jnp.float32),
        mesh=mesh,
        scratch_types=[
            pltpu.VMEM((CHUNK,), jnp.int32),
            pltpu.VMEM((CHUNK,), jnp.int32),
            pltpu.VMEM((CHUNK, H), jnp.float32),
            pltpu.VMEM((REM,), jnp.int32),
            pltpu.VMEM((REM,), jnp.int32),
            pltpu.VMEM((REM, H), jnp.float32),
            pltpu.VMEM((16, H), jnp.float32),
            pltpu.VMEM_SHARED((N, H), jnp.float32),
            pltpu.SemaphoreType.DMA,
        ],
    )
    return deg_call, agg_call


def _agg_body(src_hbm, dst_hbm, g_hbm, t_hbm,
              srcb, dstb, rows, srcr, dstr, rowsr, zbuf, t_sh, sem):
    c = lax.axis_index("c")
    s = lax.axis_index("s")
    zeros16 = jnp.zeros((16,), jnp.float32)

    for r in range(16):
        for k in range(H // 16):
            zbuf[r, pl.ds(k * 16, 16)] = zeros16

    @pl.when(s < NS - 1)
    def _():
        def zc(k, carry):
            pltpu.sync_copy(zbuf, t_sh.at[pl.ds(s * 640 + k * 16, 16)])
            return carry
        lax.fori_loop(0, 40, zc, None)

    @pl.when(s == NS - 1)
    def _():
        def zc(k, carry):
            pltpu.sync_copy(zbuf, t_sh.at[pl.ds(9600 + k * 16, 16)])
            return carry
        lax.fori_loop(0, 25, zc, None)
    plsc.subcore_barrier()

    coff = c * N
    base0 = s * EPT_AGG

    def chunk(k, carry):
        base = base0 + k * CHUNK
        pltpu.sync_copy(src_hbm.at[pl.ds(base, CHUNK)], srcb)
        pltpu.sync_copy(dst_hbm.at[pl.ds(base, CHUNK)], dstb)
        for j in range(CHUNK // 16):
            srcb[pl.ds(j * 16, 16)] = srcb[pl.ds(j * 16, 16)] + coff
        pltpu.async_copy(g_hbm.at[srcb], rows, sem).wait()
        pltpu.sync_copy(rows, t_sh.at[dstb], add=True)
        return carry
    lax.fori_loop(0, NFULL, chunk, None)

    base = base0 + NFULL * CHUNK
    pltpu.sync_copy(src_hbm.at[pl.ds(base, REM)], srcr)
    pltpu.sync_copy(dst_hbm.at[pl.ds(base, REM)], dstr)
    srcr[...] = srcr[...] + coff
    pltpu.async_copy(g_hbm.at[srcr], rowsr, sem).wait()
    pltpu.sync_copy(rowsr, t_sh.at[dstr], add=True)

    plsc.subcore_barrier()

    @pl.when(s < NS - 1)
    def _():
        pltpu.sync_copy(t_sh.at[pl.ds(s * 640, 640)],
                        t_hbm.at[pl.ds(c * N + s * 640, 640)])

    @pl.when(s == NS - 1)
    def _():
        pltpu.sync_copy(t_sh.at[pl.ds(9600, 400)],
                        t_hbm.at[pl.ds(c * N + 9600, 400)])


def _tc1(d0_ref, d1_ref, x_ref, w_ref, g_ref):
    dinv = lax.rsqrt(d0_ref[...] + d1_ref[...] + 1.0)
    xb = x_ref[...]
    nrm = jnp.sqrt(jnp.sum(xb * xb, axis=1, keepdims=True))
    hs = xb * (dinv / (nrm + 1e-8))
    g_ref[...] = jnp.dot(hs, w_ref[...], preferred_element_type=jnp.float32)


def _tc2(d0_ref, d1_ref, ta, tb, ga, gb, b1_ref, w_ref, g_ref):
    dinv = lax.rsqrt(d0_ref[...] + d1_ref[...] + 1.0)
    u0 = jnp.maximum(dinv * (ta[...] + ga[...]) + b1_ref[0][None, :], 0.0)
    u1 = jnp.maximum(dinv * (tb[...] + gb[...]) + b1_ref[1][None, :], 0.0)
    h = jnp.concatenate([u0, u1], axis=1)
    nrm = jnp.sqrt(jnp.sum(h * h, axis=1, keepdims=True))
    hs = h * (dinv / (nrm + 1e-8))
    g_ref[...] = jnp.dot(hs, w_ref[...], preferred_element_type=jnp.float32)


def _tc3(d0_ref, d1_ref, t_ref, g_ref, b2_ref, out_ref):
    p = pl.program_id(0)
    dinv = lax.rsqrt(d0_ref[...] + d1_ref[...] + 1.0)
    b = jnp.where(p == 0, b2_ref[0], b2_ref[1])
    out_ref[...] = jnp.maximum(dinv * (t_ref[...] + g_ref[...])
                               + b[None, :], 0.0)


_dspec = pl.BlockSpec((R, 1), lambda p, i: (i, 0))

_tc1_call = pl.pallas_call(
    _tc1,
    grid=(2, NBLK),
    in_specs=[
        _dspec,
        _dspec,
        pl.BlockSpec((R, D_IN), lambda p, i: (i, 0)),
        pl.BlockSpec((D_IN, H), lambda p, i: (0, p)),
    ],
    out_specs=pl.BlockSpec((R, H), lambda p, i: (p * NBLK + i, 0)),
    out_shape=jax.ShapeDtypeStruct((NC * N, H), jnp.float32),
)

_tc2_call = pl.pallas_call(
    _tc2,
    grid=(2, NBLK),
    in_specs=[
        _dspec,
        _dspec,
        pl.BlockSpec((R, H), lambda p, i: (i, 0)),
        pl.BlockSpec((R, H), lambda p, i: (NBLK + i, 0)),
        pl.BlockSpec((R, H), lambda p, i: (i, 0)),
        pl.BlockSpec((R, H), lambda p, i: (NBLK + i, 0)),
        pl.BlockSpec((2, H), lambda p, i: (0, 0)),
        pl.BlockSpec((D_HID, H), lambda p, i: (0, p)),
    ],
    out_specs=pl.BlockSpec((R, H), lambda p, i: (p * NBLK + i, 0)),
    out_shape=jax.ShapeDtypeStruct((NC * N, H), jnp.float32),
)

_tc3_call = pl.pallas_call(
    _tc3,
    grid=(2, NBLK),
    in_specs=[
        _dspec,
        _dspec,
        pl.BlockSpec((R, H), lambda p, i: (p * NBLK + i, 0)),
        pl.BlockSpec((R, H), lambda p, i: (p * NBLK + i, 0)),
        pl.BlockSpec((2, H), lambda p, i: (0, 0)),
    ],
    out_specs=pl.BlockSpec((R, H), lambda p, i: (i, p)),
    out_shape=jax.ShapeDtypeStruct((N, D_HID), jnp.float32),
)


def kernel(x, edge_index, W1, b1, W2, b2):
    deg_call, agg_call = _sc_calls()
    src = edge_index[0]
    dst = edge_index[1]
    degp = deg_call(dst)
    d0 = degp[:N].reshape(N, 1)
    d1 = degp[NPAD:NPAD + N].reshape(N, 1)
    g1 = _tc1_call(d0, d1, x, W1)
    t1 = agg_call(src, dst, g1)
    g2 = _tc2_call(d0, d1, t1, t1, g1, g1, b1.reshape(2, H), W2)
    t2 = agg_call(src, dst, g2)
    return _tc3_call(d0, d1, t2, g2, b2.reshape(2, H))


# speedup vs baseline: 18.2029x; 1.7724x over previous
import functools

import jax
import jax.numpy as jnp
from jax import lax
from jax.experimental import pallas as pl
from jax.experimental.pallas import tpu as pltpu
from jax.experimental.pallas import tpu_sc as plsc

N = 10000
E = 160000
D_IN = 296
D_HID = 256
H = 128
NC = 2
NS = 16
NPAD = 10240
SL = NPAD // NS
EPT_DEG = E // (NC * NS)
EPT_AGG = E // NS
CHUNK = 128
NFULL = EPT_AGG // CHUNK
REM = EPT_AGG - NFULL * CHUNK
R = 2000
NBLK = N // R


def _deg_body(dst_hbm, degp_hbm, dst_v, deg_v, tmp_v, slots):
    c = lax.axis_index("c")
    s = lax.axis_index("s")
    zeros16 = jnp.zeros((16,), jnp.float32)
    ones16 = jnp.ones((16,), jnp.float32)
    lane = jax.lax.iota(jnp.int32, 16)

    def zloop(i, carry):
        deg_v[pl.ds(i * 16, 16)] = zeros16
        return carry
    lax.fori_loop(0, NPAD // 16, zloop, None)

    base = (c * NS + s) * EPT_DEG
    pltpu.sync_copy(dst_hbm.at[pl.ds(base, EPT_DEG)], dst_v.at[pl.ds(0, EPT_DEG)])

    def hloop(i, carry):
        idx = dst_v[pl.ds(i * 16, 16)]
        plsc.addupdate_scatter(deg_v, [idx], ones16)
        return carry
    nfull = EPT_DEG // 16
    lax.fori_loop(0, nfull, hloop, None)
    rem_mask = lane < (EPT_DEG - nfull * 16)
    idx = dst_v[pl.ds(nfull * 16, 16)]
    idx = jnp.where(rem_mask, idx, 0)
    plsc.addupdate_scatter(deg_v, [idx], ones16, mask=rem_mask)

    pltpu.sync_copy(deg_v, slots.at[s])
    plsc.subcore_barrier()
    for j in range(NS):
        pltpu.sync_copy(slots.at[j, pl.ds(s * SL, SL)], tmp_v)
        if j == 0:
            def cploop(k, carry):
                deg_v[pl.ds(k * 16, 16)] = tmp_v[pl.ds(k * 16, 16)]
                return carry
            lax.fori_loop(0, SL // 16, cploop, None)
        else:
            def adloop(k, carry):
                deg_v[pl.ds(k * 16, 16)] = (deg_v[pl.ds(k * 16, 16)]
                                            + tmp_v[pl.ds(k * 16, 16)])
                return carry
            lax.fori_loop(0, SL // 16, adloop, None)
    pltpu.sync_copy(deg_v.at[pl.ds(0, SL)],
                    degp_hbm.at[pl.ds(c * NPAD + s * SL, SL)])


@functools.cache
def _sc_calls():
    mesh = plsc.VectorSubcoreMesh(core_axis_name="c", subcore_axis_name="s",
                                  num_cores=NC, num_subcores=NS)
    deg_call = pl.kernel(
        _deg_body,
        out_type=jax.ShapeDtypeStruct((NC * NPAD,), jnp.float32),
        mesh=mesh,
        compiler_params=pltpu.CompilerParams(needs_layout_passes=False),
        scratch_types=[
            pltpu.VMEM((EPT_DEG + 16,), jnp.int32),
            pltpu.VMEM((NPAD,), jnp.float32),
            pltpu.VMEM((SL,), jnp.float32),
            pltpu.VMEM_SHARED((NS, NPAD), jnp.float32),
        ],
    )
    agg_call = pl.kernel(
        _agg_body,
        out_type=jax.ShapeDtypeStruct((NC * N, H), jnp.float32),
        mesh=mesh,
        compiler_params=pltpu.CompilerParams(needs_layout_passes=False),
        scratch_types=[
            pltpu.VMEM((ROWS_BIG // 2, CHUNK), jnp.int32),
            pltpu.VMEM((ROWS_BIG // 2, CHUNK), jnp.int32),
            pltpu.VMEM((CHUNK, H), jnp.float32),
            pltpu.VMEM((CHUNK, H), jnp.float32),
            pltpu.VMEM((16, H), jnp.float32),
            pltpu.VMEM_SHARED((N, H), jnp.float32),
            pltpu.SemaphoreType.DMA,
            pltpu.SemaphoreType.DMA,
        ],
    )
    return deg_call, agg_call


EROWS = E // CHUNK
EROWS_PAD = 1256
ROWS_BIG = 80
ROWS_LAST = EROWS - (NS - 1) * ROWS_BIG


def _agg_body(src_hbm, dst_hbm, g_hbm, t_hbm,
              src_l, dst_l, buf_a, buf_b, zbuf, t_sh, sem_a, sem_b):
    c = lax.axis_index("c")
    s = lax.axis_index("s")
    zeros16 = jnp.zeros((16,), jnp.float32)
    base_r = c * EROWS_PAD + s * ROWS_BIG

    for r in range(16):
        for k in range(H // 16):
            zbuf[r, pl.ds(k * 16, 16)] = zeros16

    @pl.when(s < NS - 1)
    def _():
        def zc(k, carry):
            pltpu.sync_copy(zbuf, t_sh.at[pl.ds(s * 640 + k * 16, 16)])
            return carry
        lax.fori_loop(0, 40, zc, None)

    @pl.when(s == NS - 1)
    def _():
        def zc(k, carry):
            pltpu.sync_copy(zbuf, t_sh.at[pl.ds(9600 + k * 16, 16)])
            return carry
        lax.fori_loop(0, 25, zc, None)
    plsc.subcore_barrier()

    HR = ROWS_BIG // 2

    def _half(h, nproc, nload):
        pltpu.sync_copy(src_hbm.at[pl.ds(base_r + h * HR, nload)],
                        src_l.at[pl.ds(0, nload)])
        pltpu.sync_copy(dst_hbm.at[pl.ds(s * ROWS_BIG + h * HR, nload)],
                        dst_l.at[pl.ds(0, nload)])
        pltpu.async_copy(g_hbm.at[src_l.at[0]], buf_a, sem_a)

        def pair(i, carry):
            r0 = 2 * i
            desc_b = pltpu.async_copy(g_hbm.at[src_l.at[r0 + 1]], buf_b,
                                      sem_b)
            pltpu.make_async_copy(g_hbm.at[src_l.at[r0]], buf_a, sem_a).wait()
            pltpu.sync_copy(buf_a, t_sh.at[dst_l.at[r0]], add=True)

            @pl.when(i < nproc // 2 - 1)
            def _():
                pltpu.async_copy(g_hbm.at[src_l.at[r0 + 2]], buf_a, sem_a)
            desc_b.wait()
            pltpu.sync_copy(buf_b, t_sh.at[dst_l.at[r0 + 1]], add=True)
            return carry
        lax.fori_loop(0, nproc // 2, pair, None)

    @pl.when(s < NS - 1)
    def _():
        _half(0, HR, HR)
        _half(1, HR, HR)

    @pl.when(s == NS - 1)
    def _():
        _half(0, HR, HR)
        _half(1, ROWS_LAST - HR, 16)

    plsc.subcore_barrier()

    @pl.when(s < NS - 1)
    def _():
        pltpu.sync_copy(t_sh.at[pl.ds(s * 640, 640)],
                        t_hbm.at[pl.ds(c * N + s * 640, 640)])

    @pl.when(s == NS - 1)
    def _():
        pltpu.sync_copy(t_sh.at[pl.ds(9600, 400)],
                        t_hbm.at[pl.ds(c * N + 9600, 400)])


def _tc1(d0_ref, d1_ref, x_ref, w_ref, g_ref):
    dinv = lax.rsqrt(d0_ref[...] + d1_ref[...] + 1.0)
    xb = x_ref[...]
    nrm = jnp.sqrt(jnp.sum(xb * xb, axis=1, keepdims=True))
    hs = xb * (dinv / (nrm + 1e-8))
    g_ref[...] = jnp.dot(hs, w_ref[...], preferred_element_type=jnp.float32)


def _tc2(d0_ref, d1_ref, ta, tb, ga, gb, b1_ref, w_ref, g_ref):
    dinv = lax.rsqrt(d0_ref[...] + d1_ref[...] + 1.0)
    u0 = jnp.maximum(dinv * (ta[...] + ga[...]) + b1_ref[0][None, :], 0.0)
    u1 = jnp.maximum(dinv * (tb[...] + gb[...]) + b1_ref[1][None, :], 0.0)
    h = jnp.concatenate([u0, u1], axis=1)
    nrm = jnp.sqrt(jnp.sum(h * h, axis=1, keepdims=True))
    hs = h * (dinv / (nrm + 1e-8))
    g_ref[...] = jnp.dot(hs, w_ref[...], preferred_element_type=jnp.float32)


def _tc3(d0_ref, d1_ref, t_ref, g_ref, b2_ref, out_ref):
    p = pl.program_id(0)
    dinv = lax.rsqrt(d0_ref[...] + d1_ref[...] + 1.0)
    b = jnp.where(p == 0, b2_ref[0], b2_ref[1])
    out_ref[...] = jnp.maximum(dinv * (t_ref[...] + g_ref[...])
                               + b[None, :], 0.0)


_dspec = pl.BlockSpec((R, 1), lambda p, i: (i, 0))

_tc1_call = pl.pallas_call(
    _tc1,
    grid=(2, NBLK),
    in_specs=[
        _dspec,
        _dspec,
        pl.BlockSpec((R, D_IN), lambda p, i: (i, 0)),
        pl.BlockSpec((D_IN, H), lambda p, i: (0, p)),
    ],
    out_specs=pl.BlockSpec((R, H), lambda p, i: (p * NBLK + i, 0)),
    out_shape=jax.ShapeDtypeStruct((NC * N, H), jnp.float32),
)

_tc2_call = pl.pallas_call(
    _tc2,
    grid=(2, NBLK),
    in_specs=[
        _dspec,
        _dspec,
        pl.BlockSpec((R, H), lambda p, i: (i, 0)),
        pl.BlockSpec((R, H), lambda p, i: (NBLK + i, 0)),
        pl.BlockSpec((R, H), lambda p, i: (i, 0)),
        pl.BlockSpec((R, H), lambda p, i: (NBLK + i, 0)),
        pl.BlockSpec((2, H), lambda p, i: (0, 0)),
        pl.BlockSpec((D_HID, H), lambda p, i: (0, p)),
    ],
    out_specs=pl.BlockSpec((R, H), lambda p, i: (p * NBLK + i, 0)),
    out_shape=jax.ShapeDtypeStruct((NC * N, H), jnp.float32),
)

_tc3_call = pl.pallas_call(
    _tc3,
    grid=(2, NBLK),
    in_specs=[
        _dspec,
        _dspec,
        pl.BlockSpec((R, H), lambda p, i: (p * NBLK + i, 0)),
        pl.BlockSpec((R, H), lambda p, i: (p * NBLK + i, 0)),
        pl.BlockSpec((2, H), lambda p, i: (0, 0)),
    ],
    out_specs=pl.BlockSpec((R, H), lambda p, i: (i, p)),
    out_shape=jax.ShapeDtypeStruct((N, D_HID), jnp.float32),
)


def kernel(x, edge_index, W1, b1, W2, b2):
    deg_call, agg_call = _sc_calls()
    src = edge_index[0]
    dst = edge_index[1]
    src_stack = jnp.stack([src, src + N])
    src_stack = jnp.pad(src_stack, ((0, 0), (0, EROWS_PAD * CHUNK - E)))
    src2d = src_stack.reshape(NC * EROWS_PAD, CHUNK)
    dst2d = jnp.pad(dst, (0, EROWS_PAD * CHUNK - E)).reshape(EROWS_PAD, CHUNK)
    degp = deg_call(dst)
    d0 = degp[:N].reshape(N, 1)
    d1 = degp[NPAD:NPAD + N].reshape(N, 1)
    g1 = _tc1_call(d0, d1, x, W1)
    t1 = agg_call(src2d, dst2d, g1)
    g2 = _tc2_call(d0, d1, t1, t1, g1, g1, b1.reshape(2, H), W2)
    t2 = agg_call(src2d, dst2d, g2)
    return _tc3_call(d0, d1, t2, g2, b2.reshape(2, H))


# speedup vs baseline: 18.3037x; 1.0055x over previous
import functools

import jax
import jax.numpy as jnp
from jax import lax
from jax.experimental import pallas as pl
from jax.experimental.pallas import tpu as pltpu
from jax.experimental.pallas import tpu_sc as plsc

N = 10000
E = 160000
D_IN = 296
D_HID = 256
H = 128
NC = 2
NS = 16
NPAD = 10240
SL = NPAD // NS
EPT_DEG = E // (NC * NS)
EPT_AGG = E // NS
CHUNK = 128
NFULL = EPT_AGG // CHUNK
REM = EPT_AGG - NFULL * CHUNK
R = 2000
NBLK = N // R


def _deg_body(dst_hbm, degp_hbm, dst_v, deg_v, tmp_v, slots):
    c = lax.axis_index("c")
    s = lax.axis_index("s")
    zeros16 = jnp.zeros((16,), jnp.float32)
    ones16 = jnp.ones((16,), jnp.float32)
    lane = jax.lax.iota(jnp.int32, 16)

    def zloop(i, carry):
        deg_v[pl.ds(i * 16, 16)] = zeros16
        return carry
    lax.fori_loop(0, NPAD // 16, zloop, None)

    base = (c * NS + s) * EPT_DEG
    pltpu.sync_copy(dst_hbm.at[pl.ds(base, EPT_DEG)], dst_v.at[pl.ds(0, EPT_DEG)])

    def hloop(i, carry):
        idx = dst_v[pl.ds(i * 16, 16)]
        plsc.addupdate_scatter(deg_v, [idx], ones16)
        return carry
    nfull = EPT_DEG // 16
    lax.fori_loop(0, nfull, hloop, None)
    rem_mask = lane < (EPT_DEG - nfull * 16)
    idx = dst_v[pl.ds(nfull * 16, 16)]
    idx = jnp.where(rem_mask, idx, 0)
    plsc.addupdate_scatter(deg_v, [idx], ones16, mask=rem_mask)

    pltpu.sync_copy(deg_v, slots.at[s])
    plsc.subcore_barrier()
    for j in range(NS):
        pltpu.sync_copy(slots.at[j, pl.ds(s * SL, SL)], tmp_v)
        if j == 0:
            def cploop(k, carry):
                deg_v[pl.ds(k * 16, 16)] = tmp_v[pl.ds(k * 16, 16)]
                return carry
            lax.fori_loop(0, SL // 16, cploop, None)
        else:
            def adloop(k, carry):
                deg_v[pl.ds(k * 16, 16)] = (deg_v[pl.ds(k * 16, 16)]
                                            + tmp_v[pl.ds(k * 16, 16)])
                return carry
            lax.fori_loop(0, SL // 16, adloop, None)
    pltpu.sync_copy(deg_v.at[pl.ds(0, SL)],
                    degp_hbm.at[pl.ds(c * NPAD + s * SL, SL)])


@functools.cache
def _sc_calls():
    mesh = plsc.VectorSubcoreMesh(core_axis_name="c", subcore_axis_name="s",
                                  num_cores=NC, num_subcores=NS)
    deg_call = pl.kernel(
        _deg_body,
        out_type=jax.ShapeDtypeStruct((NC * NPAD,), jnp.float32),
        mesh=mesh,
        compiler_params=pltpu.CompilerParams(needs_layout_passes=False),
        scratch_types=[
            pltpu.VMEM((EPT_DEG + 16,), jnp.int32),
            pltpu.VMEM((NPAD,), jnp.float32),
            pltpu.VMEM((SL,), jnp.float32),
            pltpu.VMEM_SHARED((NS, NPAD), jnp.float32),
        ],
    )
    agg_call = pl.kernel(
        _agg_body,
        out_type=jax.ShapeDtypeStruct((NC * N, H), jnp.float32),
        mesh=mesh,
        compiler_params=pltpu.CompilerParams(needs_layout_passes=False),
        scratch_types=[
            pltpu.VMEM((ROWS_BIG // 2, CHUNK), jnp.int32),
            pltpu.VMEM((ROWS_BIG // 2, CHUNK), jnp.int32),
            pltpu.VMEM((CHUNK, H), jnp.float32),
            pltpu.VMEM((CHUNK, H), jnp.float32),
            pltpu.VMEM((16, H), jnp.float32),
            pltpu.VMEM_SHARED((N, H), jnp.float32),
            pltpu.SemaphoreType.DMA,
            pltpu.SemaphoreType.DMA,
        ],
    )
    return deg_call, agg_call


EROWS = E // CHUNK
EROWS_PAD = 1256
ROWS_BIG = 80
ROWS_LAST = EROWS - (NS - 1) * ROWS_BIG


def _agg_body(src_hbm, dst_hbm, g_hbm, t_hbm,
              src_l, dst_l, buf_a, buf_b, zbuf, t_sh, sem_a, sem_b):
    c = lax.axis_index("c")
    s = lax.axis_index("s")
    zeros16 = jnp.zeros((16,), jnp.float32)
    base_r = c * EROWS_PAD + s * ROWS_BIG

    for r in range(16):
        for k in range(H // 16):
            zbuf[r, pl.ds(k * 16, 16)] = zeros16

    @pl.when(s < NS - 1)
    def _():
        descs = [pltpu.async_copy(zbuf, t_sh.at[pl.ds(s * 640 + k * 16, 16)],
                                  sem_a) for k in range(40)]
        for d in descs:
            d.wait()

    @pl.when(s == NS - 1)
    def _():
        descs = [pltpu.async_copy(zbuf, t_sh.at[pl.ds(9600 + k * 16, 16)],
                                  sem_a) for k in range(25)]
        for d in descs:
            d.wait()
    plsc.subcore_barrier()

    HR = ROWS_BIG // 2

    def _half(h, nproc, nload):
        pltpu.sync_copy(src_hbm.at[pl.ds(base_r + h * HR, nload)],
                        src_l.at[pl.ds(0, nload)])
        pltpu.sync_copy(dst_hbm.at[pl.ds(s * ROWS_BIG + h * HR, nload)],
                        dst_l.at[pl.ds(0, nload)])
        pltpu.async_copy(g_hbm.at[src_l.at[0]], buf_a, sem_a)

        def pair(i, carry):
            r0 = 2 * i
            desc_b = pltpu.async_copy(g_hbm.at[src_l.at[r0 + 1]], buf_b,
                                      sem_b)
            pltpu.make_async_copy(g_hbm.at[src_l.at[r0]], buf_a, sem_a).wait()
            pltpu.sync_copy(buf_a, t_sh.at[dst_l.at[r0]], add=True)

            @pl.when(i < nproc // 2 - 1)
            def _():
                pltpu.async_copy(g_hbm.at[src_l.at[r0 + 2]], buf_a, sem_a)
            desc_b.wait()
            pltpu.sync_copy(buf_b, t_sh.at[dst_l.at[r0 + 1]], add=True)
            return carry
        lax.fori_loop(0, nproc // 2, pair, None)

    @pl.when(s < NS - 1)
    def _():
        _half(0, HR, HR)
        _half(1, HR, HR)

    @pl.when(s == NS - 1)
    def _():
        _half(0, HR, HR)
        _half(1, ROWS_LAST - HR, 16)

    plsc.subcore_barrier()

    @pl.when(s < NS - 1)
    def _():
        pltpu.sync_copy(t_sh.at[pl.ds(s * 640, 640)],
                        t_hbm.at[pl.ds(c * N + s * 640, 640)])

    @pl.when(s == NS - 1)
    def _():
        pltpu.sync_copy(t_sh.at[pl.ds(9600, 400)],
                        t_hbm.at[pl.ds(c * N + 9600, 400)])


def _tc1(d0_ref, d1_ref, x_ref, w_ref, g_ref):
    dinv = lax.rsqrt(d0_ref[...] + d1_ref[...] + 1.0)
    xb = x_ref[...]
    nrm = jnp.sqrt(jnp.sum(xb * xb, axis=1, keepdims=True))
    hs = xb * (dinv / (nrm + 1e-8))
    g_ref[...] = jnp.dot(hs, w_ref[...], preferred_element_type=jnp.float32)


def _tc2(d0_ref, d1_ref, ta, tb, ga, gb, b1_ref, w_ref, g_ref):
    dinv = lax.rsqrt(d0_ref[...] + d1_ref[...] + 1.0)
    u0 = jnp.maximum(dinv * (ta[...] + ga[...]) + b1_ref[0][None, :], 0.0)
    u1 = jnp.maximum(dinv * (tb[...] + gb[...]) + b1_ref[1][None, :], 0.0)
    h = jnp.concatenate([u0, u1], axis=1)
    nrm = jnp.sqrt(jnp.sum(h * h, axis=1, keepdims=True))
    hs = h * (dinv / (nrm + 1e-8))
    g_ref[...] = jnp.dot(hs, w_ref[...], preferred_element_type=jnp.float32)


def _tc3(d0_ref, d1_ref, t_ref, g_ref, b2_ref, out_ref):
    p = pl.program_id(0)
    dinv = lax.rsqrt(d0_ref[...] + d1_ref[...] + 1.0)
    b = jnp.where(p == 0, b2_ref[0], b2_ref[1])
    out_ref[...] = jnp.maximum(dinv * (t_ref[...] + g_ref[...])
                               + b[None, :], 0.0)


_dspec = pl.BlockSpec((R, 1), lambda p, i: (i, 0))

_tc1_call = pl.pallas_call(
    _tc1,
    grid=(2, NBLK),
    in_specs=[
        _dspec,
        _dspec,
        pl.BlockSpec((R, D_IN), lambda p, i: (i, 0)),
        pl.BlockSpec((D_IN, H), lambda p, i: (0, p)),
    ],
    out_specs=pl.BlockSpec((R, H), lambda p, i: (p * NBLK + i, 0)),
    out_shape=jax.ShapeDtypeStruct((NC * N, H), jnp.float32),
)

_tc2_call = pl.pallas_call(
    _tc2,
    grid=(2, NBLK),
    in_specs=[
        _dspec,
        _dspec,
        pl.BlockSpec((R, H), lambda p, i: (i, 0)),
        pl.BlockSpec((R, H), lambda p, i: (NBLK + i, 0)),
        pl.BlockSpec((R, H), lambda p, i: (i, 0)),
        pl.BlockSpec((R, H), lambda p, i: (NBLK + i, 0)),
        pl.BlockSpec((2, H), lambda p, i: (0, 0)),
        pl.BlockSpec((D_HID, H), lambda p, i: (0, p)),
    ],
    out_specs=pl.BlockSpec((R, H), lambda p, i: (p * NBLK + i, 0)),
    out_shape=jax.ShapeDtypeStruct((NC * N, H), jnp.float32),
)

_tc3_call = pl.pallas_call(
    _tc3,
    grid=(2, NBLK),
    in_specs=[
        _dspec,
        _dspec,
        pl.BlockSpec((R, H), lambda p, i: (p * NBLK + i, 0)),
        pl.BlockSpec((R, H), lambda p, i: (p * NBLK + i, 0)),
        pl.BlockSpec((2, H), lambda p, i: (0, 0)),
    ],
    out_specs=pl.BlockSpec((R, H), lambda p, i: (i, p)),
    out_shape=jax.ShapeDtypeStruct((N, D_HID), jnp.float32),
)


def kernel(x, edge_index, W1, b1, W2, b2):
    deg_call, agg_call = _sc_calls()
    src = edge_index[0]
    dst = edge_index[1]
    src_stack = jnp.stack([src, src + N])
    src_stack = jnp.pad(src_stack, ((0, 0), (0, EROWS_PAD * CHUNK - E)))
    src2d = src_stack.reshape(NC * EROWS_PAD, CHUNK)
    dst2d = jnp.pad(dst, (0, EROWS_PAD * CHUNK - E)).reshape(EROWS_PAD, CHUNK)
    degp = deg_call(dst)
    d0 = degp[:N].reshape(N, 1)
    d1 = degp[NPAD:NPAD + N].reshape(N, 1)
    g1 = _tc1_call(d0, d1, x, W1)
    t1 = agg_call(src2d, dst2d, g1)
    g2 = _tc2_call(d0, d1, t1, t1, g1, g1, b1.reshape(2, H), W2)
    t2 = agg_call(src2d, dst2d, g2)
    return _tc3_call(d0, d1, t2, g2, b2.reshape(2, H))


# speedup vs baseline: 18.3664x; 1.0034x over previous
import functools

import jax
import jax.numpy as jnp
from jax import lax
from jax.experimental import pallas as pl
from jax.experimental.pallas import tpu as pltpu
from jax.experimental.pallas import tpu_sc as plsc

N = 10000
E = 160000
D_IN = 296
D_HID = 256
H = 128
NC = 2
NS = 16
NPAD = 10240
SL = NPAD // NS
EPT_DEG = E // (NC * NS)
EPT_AGG = E // NS
CHUNK = 128
NFULL = EPT_AGG // CHUNK
REM = EPT_AGG - NFULL * CHUNK
R = 2000
NBLK = N // R


def _deg_body(dst_hbm, degp_hbm, dst_v, deg_v, tmp_v, slots):
    c = lax.axis_index("c")
    s = lax.axis_index("s")
    zeros16 = jnp.zeros((16,), jnp.float32)
    ones16 = jnp.ones((16,), jnp.float32)
    lane = jax.lax.iota(jnp.int32, 16)

    def zloop(i, carry):
        deg_v[pl.ds(i * 16, 16)] = zeros16
        return carry
    lax.fori_loop(0, NPAD // 16, zloop, None)

    base = (c * NS + s) * EPT_DEG
    pltpu.sync_copy(dst_hbm.at[pl.ds(base, EPT_DEG)], dst_v.at[pl.ds(0, EPT_DEG)])

    def hloop(i, carry):
        idx = dst_v[pl.ds(i * 16, 16)]
        plsc.addupdate_scatter(deg_v, [idx], ones16)
        return carry
    nfull = EPT_DEG // 16
    lax.fori_loop(0, nfull, hloop, None)
    rem_mask = lane < (EPT_DEG - nfull * 16)
    idx = dst_v[pl.ds(nfull * 16, 16)]
    idx = jnp.where(rem_mask, idx, 0)
    plsc.addupdate_scatter(deg_v, [idx], ones16, mask=rem_mask)

    pltpu.sync_copy(deg_v, slots.at[s])
    plsc.subcore_barrier()
    for j in range(NS):
        pltpu.sync_copy(slots.at[j, pl.ds(s * SL, SL)], tmp_v)
        if j == 0:
            def cploop(k, carry):
                deg_v[pl.ds(k * 16, 16)] = tmp_v[pl.ds(k * 16, 16)]
                return carry
            lax.fori_loop(0, SL // 16, cploop, None)
        else:
            def adloop(k, carry):
                deg_v[pl.ds(k * 16, 16)] = (deg_v[pl.ds(k * 16, 16)]
                                            + tmp_v[pl.ds(k * 16, 16)])
                return carry
            lax.fori_loop(0, SL // 16, adloop, None)
    pltpu.sync_copy(deg_v.at[pl.ds(0, SL)],
                    degp_hbm.at[pl.ds(c * NPAD + s * SL, SL)])


@functools.cache
def _sc_calls():
    mesh = plsc.VectorSubcoreMesh(core_axis_name="c", subcore_axis_name="s",
                                  num_cores=NC, num_subcores=NS)
    deg_call = pl.kernel(
        _deg_body,
        out_type=jax.ShapeDtypeStruct((NC * NPAD,), jnp.float32),
        mesh=mesh,
        compiler_params=pltpu.CompilerParams(needs_layout_passes=False),
        scratch_types=[
            pltpu.VMEM((EPT_DEG + 16,), jnp.int32),
            pltpu.VMEM((NPAD,), jnp.float32),
            pltpu.VMEM((SL,), jnp.float32),
            pltpu.VMEM_SHARED((NS, NPAD), jnp.float32),
        ],
    )
    agg_call = pl.kernel(
        _agg_body,
        out_type=jax.ShapeDtypeStruct((NC * N, H), jnp.float32),
        mesh=mesh,
        compiler_params=pltpu.CompilerParams(needs_layout_passes=False),
        scratch_types=[
            pltpu.VMEM((ROWS_BIG // 2, CHUNK), jnp.int32),
            pltpu.VMEM((ROWS_BIG // 2, CHUNK), jnp.int32),
            pltpu.VMEM((CHUNK, H), jnp.float32),
            pltpu.VMEM((CHUNK, H), jnp.float32),
            pltpu.VMEM((16, H), jnp.float32),
            pltpu.VMEM_SHARED((N, H), jnp.float32),
            pltpu.SemaphoreType.DMA,
            pltpu.SemaphoreType.DMA,
        ],
    )
    return deg_call, agg_call


EROWS = E // CHUNK
EROWS_PAD = 1256
ROWS_BIG = 80
ROWS_LAST = EROWS - (NS - 1) * ROWS_BIG


def _agg_body(src_hbm, dst_hbm, g_hbm, t_hbm,
              src_l, dst_l, buf_a, buf_b, zbuf, t_sh, sem_a, sem_b):
    c = lax.axis_index("c")
    s = lax.axis_index("s")
    zeros16 = jnp.zeros((16,), jnp.float32)
    base_r = c * EROWS_PAD + s * ROWS_BIG

    for r in range(16):
        for k in range(H // 16):
            zbuf[r, pl.ds(k * 16, 16)] = zeros16

    @pl.when(s < NS - 1)
    def _():
        descs = [pltpu.async_copy(zbuf, t_sh.at[pl.ds(s * 640 + k * 16, 16)],
                                  sem_a) for k in range(40)]
        for d in descs:
            d.wait()

    @pl.when(s == NS - 1)
    def _():
        descs = [pltpu.async_copy(zbuf, t_sh.at[pl.ds(9600 + k * 16, 16)],
                                  sem_a) for k in range(25)]
        for d in descs:
            d.wait()
    plsc.subcore_barrier()

    HR = ROWS_BIG // 2

    def _half(h, nproc, nload):
        pltpu.sync_copy(src_hbm.at[pl.ds(base_r + h * HR, nload)],
                        src_l.at[pl.ds(0, nload)])
        pltpu.sync_copy(dst_hbm.at[pl.ds(s * ROWS_BIG + h * HR, nload)],
                        dst_l.at[pl.ds(0, nload)])
        pltpu.async_copy(g_hbm.at[src_l.at[0]], buf_a, sem_a)

        def pair(i, carry):
            r0 = 2 * i
            desc_b = pltpu.async_copy(g_hbm.at[src_l.at[r0 + 1]], buf_b,
                                      sem_b)
            pltpu.make_async_copy(g_hbm.at[src_l.at[r0]], buf_a, sem_a).wait()
            pltpu.sync_copy(buf_a, t_sh.at[dst_l.at[r0]], add=True)

            @pl.when(i < nproc // 2 - 1)
            def _():
                pltpu.async_copy(g_hbm.at[src_l.at[r0 + 2]], buf_a, sem_a)
            desc_b.wait()
            pltpu.sync_copy(buf_b, t_sh.at[dst_l.at[r0 + 1]], add=True)
            return carry
        lax.fori_loop(0, nproc // 2, pair, None)

    @pl.when(s < NS - 1)
    def _():
        _half(0, HR, HR)
        _half(1, HR, HR)

    @pl.when(s == NS - 1)
    def _():
        _half(0, HR, HR)
        _half(1, ROWS_LAST - HR, 16)

    plsc.subcore_barrier()

    @pl.when(s < NS - 1)
    def _():
        pltpu.sync_copy(t_sh.at[pl.ds(s * 640, 640)],
                        t_hbm.at[pl.ds(c * N + s * 640, 640)])

    @pl.when(s == NS - 1)
    def _():
        pltpu.sync_copy(t_sh.at[pl.ds(9600, 400)],
                        t_hbm.at[pl.ds(c * N + 9600, 400)])


def _tc1(d0_ref, d1_ref, x_ref, w_ref, g_ref):
    dinv = lax.rsqrt(d0_ref[...] + d1_ref[...] + 1.0)
    xb = x_ref[...]
    nrm = jnp.sqrt(jnp.sum(xb * xb, axis=1, keepdims=True))
    hs = xb * (dinv / (nrm + 1e-8))
    g_ref[...] = jnp.dot(hs, w_ref[...], preferred_element_type=jnp.float32)


def _tc2(d0_ref, d1_ref, ta, tb, ga, gb, b1_ref, w_ref, g_ref):
    dinv = lax.rsqrt(d0_ref[...] + d1_ref[...] + 1.0)
    u0 = jnp.maximum(dinv * (ta[...] + ga[...]) + b1_ref[0][None, :], 0.0)
    u1 = jnp.maximum(dinv * (tb[...] + gb[...]) + b1_ref[1][None, :], 0.0)
    h = jnp.concatenate([u0, u1], axis=1)
    nrm = jnp.sqrt(jnp.sum(h * h, axis=1, keepdims=True))
    hs = h * (dinv / (nrm + 1e-8))
    g_ref[...] = jnp.dot(hs, w_ref[...], preferred_element_type=jnp.float32)


def _tc3(d0_ref, d1_ref, t_ref, g_ref, b2_ref, out_ref):
    p = pl.program_id(1)
    dinv = lax.rsqrt(d0_ref[...] + d1_ref[...] + 1.0)
    b = jnp.where(p == 0, b2_ref[0], b2_ref[1])
    out_ref[...] = jnp.maximum(dinv * (t_ref[...] + g_ref[...])
                               + b[None, :], 0.0)


_dspec = pl.BlockSpec((R, 1), lambda i, p: (i, 0))

_tc1_call = pl.pallas_call(
    _tc1,
    grid=(NBLK, 2),
    in_specs=[
        _dspec,
        _dspec,
        pl.BlockSpec((R, D_IN), lambda i, p: (i, 0)),
        pl.BlockSpec((D_IN, H), lambda i, p: (0, p)),
    ],
    out_specs=pl.BlockSpec((R, H), lambda i, p: (p * NBLK + i, 0)),
    out_shape=jax.ShapeDtypeStruct((NC * N, H), jnp.float32),
)

_tc2_call = pl.pallas_call(
    _tc2,
    grid=(NBLK, 2),
    in_specs=[
        _dspec,
        _dspec,
        pl.BlockSpec((R, H), lambda i, p: (i, 0)),
        pl.BlockSpec((R, H), lambda i, p: (NBLK + i, 0)),
        pl.BlockSpec((R, H), lambda i, p: (i, 0)),
        pl.BlockSpec((R, H), lambda i, p: (NBLK + i, 0)),
        pl.BlockSpec((2, H), lambda i, p: (0, 0)),
        pl.BlockSpec((D_HID, H), lambda i, p: (0, p)),
    ],
    out_specs=pl.BlockSpec((R, H), lambda i, p: (p * NBLK + i, 0)),
    out_shape=jax.ShapeDtypeStruct((NC * N, H), jnp.float32),
)

_tc3_call = pl.pallas_call(
    _tc3,
    grid=(NBLK, 2),
    in_specs=[
        _dspec,
        _dspec,
        pl.BlockSpec((R, H), lambda i, p: (p * NBLK + i, 0)),
        pl.BlockSpec((R, H), lambda i, p: (p * NBLK + i, 0)),
        pl.BlockSpec((2, H), lambda i, p: (0, 0)),
    ],
    out_specs=pl.BlockSpec((R, H), lambda i, p: (i, p)),
    out_shape=jax.ShapeDtypeStruct((N, D_HID), jnp.float32),
)


def kernel(x, edge_index, W1, b1, W2, b2):
    deg_call, agg_call = _sc_calls()
    src = edge_index[0]
    dst = edge_index[1]
    src_stack = jnp.stack([src, src + N])
    src_stack = jnp.pad(src_stack, ((0, 0), (0, EROWS_PAD * CHUNK - E)))
    src2d = src_stack.reshape(NC * EROWS_PAD, CHUNK)
    dst2d = jnp.pad(dst, (0, EROWS_PAD * CHUNK - E)).reshape(EROWS_PAD, CHUNK)
    degp = deg_call(dst)
    d0 = degp[:N].reshape(N, 1)
    d1 = degp[NPAD:NPAD + N].reshape(N, 1)
    g1 = _tc1_call(d0, d1, x, W1)
    t1 = agg_call(src2d, dst2d, g1)
    g2 = _tc2_call(d0, d1, t1, t1, g1, g1, b1.reshape(2, H), W2)
    t2 = agg_call(src2d, dst2d, g2)
    return _tc3_call(d0, d1, t2, g2, b2.reshape(2, H))


# speedup vs baseline: 19.3079x; 1.0513x over previous
import functools

import jax
import jax.numpy as jnp
from jax import lax
from jax.experimental import pallas as pl
from jax.experimental.pallas import tpu as pltpu
from jax.experimental.pallas import tpu_sc as plsc

N = 10000
E = 160000
D_IN = 296
D_HID = 256
H = 128
NC = 2
NS = 16
NPAD = 10240
SL = NPAD // NS
EPT_DEG = E // (NC * NS)
EPT_AGG = E // NS
CHUNK = 128
NFULL = EPT_AGG // CHUNK
REM = EPT_AGG - NFULL * CHUNK
R = 2000
NBLK = N // R


def _deg_body(edge_hbm, degp_hbm, ebuf, deg_v, tmp_v, slots):
    c = lax.axis_index("c")
    s = lax.axis_index("s")
    w = c * NS + s
    zeros16 = jnp.zeros((16,), jnp.float32)
    ones16 = jnp.ones((16,), jnp.float32)

    def zloop(i, carry):
        deg_v[pl.ds(i * 16, 16)] = zeros16
        return carry
    lax.fori_loop(0, NPAD // 16, zloop, None)

    def hist(n, carry):
        idx = ebuf[1, pl.ds(n * 16, 16)]
        plsc.addupdate_scatter(deg_v, [idx], ones16)
        return carry

    @pl.when(w < 2)
    def _():
        pltpu.sync_copy(edge_hbm.at[:, pl.ds(w * 5120, 5120)], ebuf)
        lax.fori_loop(0, 320, hist, None)

    @pl.when(w >= 2)
    def _():
        base = 10240 + (w - 2) * 4992
        pltpu.sync_copy(edge_hbm.at[:, pl.ds(base, 4992)],
                        ebuf.at[:, pl.ds(0, 4992)])
        lax.fori_loop(0, 312, hist, None)

    pltpu.sync_copy(deg_v, slots.at[s])
    plsc.subcore_barrier()
    for j in range(NS):
        pltpu.sync_copy(slots.at[j, pl.ds(s * SL, SL)], tmp_v)
        if j == 0:
            def cploop(k, carry):
                deg_v[pl.ds(k * 16, 16)] = tmp_v[pl.ds(k * 16, 16)]
                return carry
            lax.fori_loop(0, SL // 16, cploop, None)
        else:
            def adloop(k, carry):
                deg_v[pl.ds(k * 16, 16)] = (deg_v[pl.ds(k * 16, 16)]
                                            + tmp_v[pl.ds(k * 16, 16)])
                return carry
            lax.fori_loop(0, SL // 16, adloop, None)
    pltpu.sync_copy(deg_v.at[pl.ds(0, SL)],
                    degp_hbm.at[pl.ds(c * NPAD + s * SL, SL)])


@functools.cache
def _sc_calls():
    mesh = plsc.VectorSubcoreMesh(core_axis_name="c", subcore_axis_name="s",
                                  num_cores=NC, num_subcores=NS)
    deg_call = pl.kernel(
        _deg_body,
        out_type=jax.ShapeDtypeStruct((NC * NPAD,), jnp.float32),
        mesh=mesh,
        compiler_params=pltpu.CompilerParams(needs_layout_passes=False),
        scratch_types=[
            pltpu.VMEM((2, 5120), jnp.int32),
            pltpu.VMEM((NPAD,), jnp.float32),
            pltpu.VMEM((SL,), jnp.float32),
            pltpu.VMEM_SHARED((NS, NPAD), jnp.float32),
        ],
    )
    agg_call = pl.kernel(
        _agg_body,
        out_type=jax.ShapeDtypeStruct((NC * N, H), jnp.float32),
        mesh=mesh,
        compiler_params=pltpu.CompilerParams(needs_layout_passes=False),
        scratch_types=[
            pltpu.VMEM((ROWS_BIG // 2, CHUNK), jnp.int32),
            pltpu.VMEM((ROWS_BIG // 2, CHUNK), jnp.int32),
            pltpu.VMEM((CHUNK, H), jnp.float32),
            pltpu.VMEM((CHUNK, H), jnp.float32),
            pltpu.VMEM((16, H), jnp.float32),
            pltpu.VMEM_SHARED((N, H), jnp.float32),
            pltpu.SemaphoreType.DMA,
            pltpu.SemaphoreType.DMA,
        ],
    )
    return deg_call, agg_call


EROWS = E // CHUNK
EROWS_PAD = 1256
ROWS_BIG = 80
ROWS_LAST = EROWS - (NS - 1) * ROWS_BIG


def _agg_body(src_hbm, dst_hbm, g_hbm, t_hbm,
              src_l, dst_l, buf_a, buf_b, zbuf, t_sh, sem_a, sem_b):
    c = lax.axis_index("c")
    s = lax.axis_index("s")
    zeros16 = jnp.zeros((16,), jnp.float32)
    base_r = c * EROWS_PAD + s * ROWS_BIG

    for r in range(16):
        for k in range(H // 16):
            zbuf[r, pl.ds(k * 16, 16)] = zeros16

    @pl.when(s < NS - 1)
    def _():
        descs = [pltpu.async_copy(zbuf, t_sh.at[pl.ds(s * 640 + k * 16, 16)],
                                  sem_a) for k in range(40)]
        for d in descs:
            d.wait()

    @pl.when(s == NS - 1)
    def _():
        descs = [pltpu.async_copy(zbuf, t_sh.at[pl.ds(9600 + k * 16, 16)],
                                  sem_a) for k in range(25)]
        for d in descs:
            d.wait()
    plsc.subcore_barrier()

    HR = ROWS_BIG // 2

    def _half(h, nproc, nload):
        pltpu.sync_copy(src_hbm.at[pl.ds(base_r + h * HR, nload)],
                        src_l.at[pl.ds(0, nload)])
        pltpu.sync_copy(dst_hbm.at[pl.ds(s * ROWS_BIG + h * HR, nload)],
                        dst_l.at[pl.ds(0, nload)])
        pltpu.async_copy(g_hbm.at[src_l.at[0]], buf_a, sem_a)

        def pair(i, carry):
            r0 = 2 * i
            desc_b = pltpu.async_copy(g_hbm.at[src_l.at[r0 + 1]], buf_b,
                                      sem_b)
            pltpu.make_async_copy(g_hbm.at[src_l.at[r0]], buf_a, sem_a).wait()
            pltpu.sync_copy(buf_a, t_sh.at[dst_l.at[r0]], add=True)

            @pl.when(i < nproc // 2 - 1)
            def _():
                pltpu.async_copy(g_hbm.at[src_l.at[r0 + 2]], buf_a, sem_a)
            desc_b.wait()
            pltpu.sync_copy(buf_b, t_sh.at[dst_l.at[r0 + 1]], add=True)
            return carry
        lax.fori_loop(0, nproc // 2, pair, None)

    @pl.when(s < NS - 1)
    def _():
        _half(0, HR, HR)
        _half(1, HR, HR)

    @pl.when(s == NS - 1)
    def _():
        _half(0, HR, HR)
        _half(1, ROWS_LAST - HR, 16)

    plsc.subcore_barrier()

    @pl.when(s < NS - 1)
    def _():
        pltpu.sync_copy(t_sh.at[pl.ds(s * 640, 640)],
                        t_hbm.at[pl.ds(c * N + s * 640, 640)])

    @pl.when(s == NS - 1)
    def _():
        pltpu.sync_copy(t_sh.at[pl.ds(9600, 400)],
                        t_hbm.at[pl.ds(c * N + 9600, 400)])


def _tc1(dv_ref, x_ref, w_ref, g_ref):
    dinv = dv_ref[:, 0:1]
    xb = x_ref[...]
    nrm = jnp.sqrt(jnp.sum(xb * xb, axis=1, keepdims=True))
    hs = xb * (dinv / (nrm + 1e-8))
    g_ref[...] = jnp.dot(hs, w_ref[...], preferred_element_type=jnp.float32)


def _tc2(dv_ref, ta, tb, ga, gb, b1_ref, w_ref, g_ref):
    dv = dv_ref[...]
    u0 = jnp.maximum(dv * (ta[...] + ga[...]) + b1_ref[0][None, :], 0.0)
    u1 = jnp.maximum(dv * (tb[...] + gb[...]) + b1_ref[1][None, :], 0.0)
    h = jnp.concatenate([u0, u1], axis=1)
    nrm = jnp.sqrt(jnp.sum(h * h, axis=1, keepdims=True))
    hs = h * (dv_ref[:, 0:1] / (nrm + 1e-8))
    g_ref[...] = jnp.dot(hs, w_ref[...], preferred_element_type=jnp.float32)


def _tc3(dv_ref, t_ref, g_ref, b2_ref, out_ref):
    p = pl.program_id(1)
    b = jnp.where(p == 0, b2_ref[0], b2_ref[1])
    out_ref[...] = jnp.maximum(dv_ref[...] * (t_ref[...] + g_ref[...])
                               + b[None, :], 0.0)


_dspec = pl.BlockSpec((R, H), lambda i, p: (i, 0))

_tc1_call = pl.pallas_call(
    _tc1,
    grid=(NBLK, 2),
    in_specs=[
        _dspec,
        pl.BlockSpec((R, D_IN), lambda i, p: (i, 0)),
        pl.BlockSpec((D_IN, H), lambda i, p: (0, p)),
    ],
    out_specs=pl.BlockSpec((R, H), lambda i, p: (p * NBLK + i, 0)),
    out_shape=jax.ShapeDtypeStruct((NC * N, H), jnp.float32),
)

_tc2_call = pl.pallas_call(
    _tc2,
    grid=(NBLK, 2),
    in_specs=[
        _dspec,
        pl.BlockSpec((R, H), lambda i, p: (i, 0)),
        pl.BlockSpec((R, H), lambda i, p: (NBLK + i, 0)),
        pl.BlockSpec((R, H), lambda i, p: (i, 0)),
        pl.BlockSpec((R, H), lambda i, p: (NBLK + i, 0)),
        pl.BlockSpec((2, H), lambda i, p: (0, 0)),
        pl.BlockSpec((D_HID, H), lambda i, p: (0, p)),
    ],
    out_specs=pl.BlockSpec((R, H), lambda i, p: (p * NBLK + i, 0)),
    out_shape=jax.ShapeDtypeStruct((NC * N, H), jnp.float32),
)

_tc3_call = pl.pallas_call(
    _tc3,
    grid=(NBLK, 2),
    in_specs=[
        _dspec,
        pl.BlockSpec((R, H), lambda i, p: (p * NBLK + i, 0)),
        pl.BlockSpec((R, H), lambda i, p: (p * NBLK + i, 0)),
        pl.BlockSpec((2, H), lambda i, p: (0, 0)),
    ],
    out_specs=pl.BlockSpec((R, H), lambda i, p: (i, p)),
    out_shape=jax.ShapeDtypeStruct((N, D_HID), jnp.float32),
)


def kernel(x, edge_index, W1, b1, W2, b2):
    deg_call, agg_call = _sc_calls()
    src = edge_index[0]
    dst = edge_index[1]
    pad_n = EROWS_PAD * CHUNK - E
    srcp = jnp.pad(src, (0, pad_n))
    src2d = jnp.concatenate([srcp, srcp + N]).reshape(NC * EROWS_PAD, CHUNK)
    dst2d = jnp.pad(dst, (0, pad_n)).reshape(EROWS_PAD, CHUNK)
    degp = deg_call(edge_index)
    dinv = lax.rsqrt(degp[:N] + degp[NPAD:NPAD + N] + 1.0)
    dv = jnp.broadcast_to(dinv[:, None], (N, H))
    g1 = _tc1_call(dv, x, W1)
    t1 = agg_call(src2d, dst2d, g1)
    g2 = _tc2_call(dv, t1, t1, g1, g1, b1.reshape(2, H), W2)
    t2 = agg_call(src2d, dst2d, g2)
    return _tc3_call(dv, t2, g2, b2.reshape(2, H))


# speedup vs baseline: 20.6553x; 1.0698x over previous
import functools

import jax
import jax.numpy as jnp
from jax import lax
from jax.experimental import pallas as pl
from jax.experimental.pallas import tpu as pltpu
from jax.experimental.pallas import tpu_sc as plsc

N = 10000
E = 160000
D_IN = 296
D_HID = 256
H = 128
NC = 2
NS = 16
NPAD = 10240
SL = NPAD // NS
CHUNK = 128
R = 2000
NBLK = N // R

EROWS = E // CHUNK
EROWS_PAD = 1256
ROWS_BIG = 80
ROWS_LAST = EROWS - (NS - 1) * ROWS_BIG
HR = ROWS_BIG // 2


def _deg_body(edge_hbm, degp_hbm, ebuf, deg_v, tmp_v, slots):
    c = lax.axis_index("c")
    s = lax.axis_index("s")
    w = c * NS + s
    zeros16 = jnp.zeros((16,), jnp.float32)
    ones16 = jnp.ones((16,), jnp.float32)

    def zloop(i, carry):
        deg_v[pl.ds(i * 16, 16)] = zeros16
        return carry
    lax.fori_loop(0, NPAD // 16, zloop, None)

    def hist(n, carry):
        idx = ebuf[1, pl.ds(n * 16, 16)]
        plsc.addupdate_scatter(deg_v, [idx], ones16)
        return carry

    @pl.when(w < 2)
    def _():
        pltpu.sync_copy(edge_hbm.at[:, pl.ds(w * 5120, 5120)], ebuf)
        lax.fori_loop(0, 320, hist, None)

    @pl.when(w >= 2)
    def _():
        base = 10240 + (w - 2) * 4992
        pltpu.sync_copy(edge_hbm.at[:, pl.ds(base, 4992)],
                        ebuf.at[:, pl.ds(0, 4992)])
        lax.fori_loop(0, 312, hist, None)

    pltpu.sync_copy(deg_v, slots.at[s])
    plsc.subcore_barrier()
    for j in range(NS):
        pltpu.sync_copy(slots.at[j, pl.ds(s * SL, SL)], tmp_v)
        if j == 0:
            def cploop(k, carry):
                deg_v[pl.ds(k * 16, 16)] = tmp_v[pl.ds(k * 16, 16)]
                return carry
            lax.fori_loop(0, SL // 16, cploop, None)
        else:
            def adloop(k, carry):
                deg_v[pl.ds(k * 16, 16)] = (deg_v[pl.ds(k * 16, 16)]
                                            + tmp_v[pl.ds(k * 16, 16)])
                return carry
            lax.fori_loop(0, SL // 16, adloop, None)
    pltpu.sync_copy(deg_v.at[pl.ds(0, SL)],
                    degp_hbm.at[pl.ds(c * NPAD + s * SL, SL)])


def _agg_body(src_hbm, dst_hbm, g0_hbm, g1_hbm, t0_hbm, t1_hbm,
              src_l, dst_l, buf_a, buf_b, zbuf, t_sh, sem_a, sem_b):
    c = lax.axis_index("c")
    s = lax.axis_index("s")
    zeros16 = jnp.zeros((16,), jnp.float32)
    base_r = s * ROWS_BIG

    for r in range(16):
        for k in range(H // 16):
            zbuf[r, pl.ds(k * 16, 16)] = zeros16

    def _zero_descs(nslabs, base):
        return [pltpu.async_copy(zbuf, t_sh.at[pl.ds(base + k * 16, 16)],
                                 sem_a) for k in range(nslabs)]

    def _load_idx(h, nload):
        pltpu.sync_copy(src_hbm.at[pl.ds(base_r + h * HR, nload)],
                        src_l.at[pl.ds(0, nload)])
        pltpu.sync_copy(dst_hbm.at[pl.ds(base_r + h * HR, nload)],
                        dst_l.at[pl.ds(0, nload)])

    @pl.when(s < NS - 1)
    def _():
        descs = _zero_descs(40, s * 640)
        _load_idx(0, HR)
        for d in descs:
            d.wait()

    @pl.when(s == NS - 1)
    def _():
        descs = _zero_descs(25, 9600)
        _load_idx(0, HR)
        for d in descs:
            d.wait()
    plsc.subcore_barrier()

    def _pipe(g_hbm, nproc):
        pltpu.async_copy(g_hbm.at[src_l.at[0]], buf_a, sem_a)

        def pair(i, carry):
            r0 = 2 * i
            desc_b = pltpu.async_copy(g_hbm.at[src_l.at[r0 + 1]], buf_b,
                                      sem_b)
            pltpu.make_async_copy(g_hbm.at[src_l.at[r0]], buf_a, sem_a).wait()
            pltpu.sync_copy(buf_a, t_sh.at[dst_l.at[r0]], add=True)

            @pl.when(i < nproc // 2 - 1)
            def _():
                pltpu.async_copy(g_hbm.at[src_l.at[r0 + 2]], buf_a, sem_a)
            desc_b.wait()
            pltpu.sync_copy(buf_b, t_sh.at[dst_l.at[r0 + 1]], add=True)
            return carry
        lax.fori_loop(0, nproc // 2, pair, None)

    def _run(g_hbm):
        @pl.when(s < NS - 1)
        def _():
            _pipe(g_hbm, HR)
            _load_idx(1, HR)
            _pipe(g_hbm, HR)

        @pl.when(s == NS - 1)
        def _():
            _pipe(g_hbm, HR)
            _load_idx(1, 16)
            _pipe(g_hbm, ROWS_LAST - HR)

    @pl.when(c == 0)
    def _():
        _run(g0_hbm)

    @pl.when(c == 1)
    def _():
        _run(g1_hbm)

    plsc.subcore_barrier()

    def _writeback(t_hbm):
        @pl.when(s < NS - 1)
        def _():
            pltpu.sync_copy(t_sh.at[pl.ds(s * 640, 640)],
                            t_hbm.at[pl.ds(s * 640, 640)])

        @pl.when(s == NS - 1)
        def _():
            pltpu.sync_copy(t_sh.at[pl.ds(9600, 400)],
                            t_hbm.at[pl.ds(9600, 400)])

    @pl.when(c == 0)
    def _():
        _writeback(t0_hbm)

    @pl.when(c == 1)
    def _():
        _writeback(t1_hbm)


@functools.cache
def _sc_calls():
    mesh = plsc.VectorSubcoreMesh(core_axis_name="c", subcore_axis_name="s",
                                  num_cores=NC, num_subcores=NS)
    deg_call = pl.kernel(
        _deg_body,
        out_type=jax.ShapeDtypeStruct((NC * NPAD,), jnp.float32),
        mesh=mesh,
        compiler_params=pltpu.CompilerParams(needs_layout_passes=False),
        scratch_types=[
            pltpu.VMEM((2, 5120), jnp.int32),
            pltpu.VMEM((NPAD,), jnp.float32),
            pltpu.VMEM((SL,), jnp.float32),
            pltpu.VMEM_SHARED((NS, NPAD), jnp.float32),
        ],
    )
    agg_call = pl.kernel(
        _agg_body,
        out_type=[jax.ShapeDtypeStruct((N, H), jnp.float32),
                  jax.ShapeDtypeStruct((N, H), jnp.float32)],
        mesh=mesh,
        compiler_params=pltpu.CompilerParams(needs_layout_passes=False),
        scratch_types=[
            pltpu.VMEM((HR, CHUNK), jnp.int32),
            pltpu.VMEM((HR, CHUNK), jnp.int32),
            pltpu.VMEM((CHUNK, H), jnp.float32),
            pltpu.VMEM((CHUNK, H), jnp.float32),
            pltpu.VMEM((16, H), jnp.float32),
            pltpu.VMEM_SHARED((N, H), jnp.float32),
            pltpu.SemaphoreType.DMA,
            pltpu.SemaphoreType.DMA,
        ],
    )
    return deg_call, agg_call


def _tc1(dv_ref, x_ref, w_ref, g0_ref, g1_ref):
    dinv = dv_ref[:, 0:1]
    xb = x_ref[...]
    nrm = jnp.sqrt(jnp.sum(xb * xb, axis=1, keepdims=True))
    hs = xb * (dinv / (nrm + 1e-8))
    g = jnp.dot(hs, w_ref[...], preferred_element_type=jnp.float32)
    g0_ref[...] = g[:, :H]
    g1_ref[...] = g[:, H:]


def _tc2(dv_ref, t0, t1, g0, g1, b1_ref, w_ref, g20_ref, g21_ref):
    dv = dv_ref[...]
    u0 = jnp.maximum(dv * (t0[...] + g0[...]) + b1_ref[0][None, :], 0.0)
    u1 = jnp.maximum(dv * (t1[...] + g1[...]) + b1_ref[1][None, :], 0.0)
    h = jnp.concatenate([u0, u1], axis=1)
    nrm = jnp.sqrt(jnp.sum(h * h, axis=1, keepdims=True))
    hs = h * (dv_ref[:, 0:1] / (nrm + 1e-8))
    g = jnp.dot(hs, w_ref[...], preferred_element_type=jnp.float32)
    g20_ref[...] = g[:, :H]
    g21_ref[...] = g[:, H:]


def _tc3(dv_ref, t0, t1, g0, g1, b2_ref, out_ref):
    dv = dv_ref[...]
    u0 = jnp.maximum(dv * (t0[...] + g0[...]) + b2_ref[0][None, :], 0.0)
    u1 = jnp.maximum(dv * (t1[...] + g1[...]) + b2_ref[1][None, :], 0.0)
    out_ref[...] = jnp.concatenate([u0, u1], axis=1)


_rspec = pl.BlockSpec((R, H), lambda i: (i, 0))

_tc1_call = pl.pallas_call(
    _tc1,
    grid=(NBLK,),
    in_specs=[
        _rspec,
        pl.BlockSpec((R, D_IN), lambda i: (i, 0)),
        pl.BlockSpec((D_IN, D_HID), lambda i: (0, 0)),
    ],
    out_specs=[_rspec, _rspec],
    out_shape=[jax.ShapeDtypeStruct((N, H), jnp.float32),
               jax.ShapeDtypeStruct((N, H), jnp.float32)],
)

_tc2_call = pl.pallas_call(
    _tc2,
    grid=(NBLK,),
    in_specs=[
        _rspec, _rspec, _rspec, _rspec, _rspec,
        pl.BlockSpec((2, H), lambda i: (0, 0)),
        pl.BlockSpec((D_HID, D_HID), lambda i: (0, 0)),
    ],
    out_specs=[_rspec, _rspec],
    out_shape=[jax.ShapeDtypeStruct((N, H), jnp.float32),
               jax.ShapeDtypeStruct((N, H), jnp.float32)],
)

_tc3_call = pl.pallas_call(
    _tc3,
    grid=(NBLK,),
    in_specs=[
        _rspec, _rspec, _rspec, _rspec, _rspec,
        pl.BlockSpec((2, H), lambda i: (0, 0)),
    ],
    out_specs=pl.BlockSpec((R, D_HID), lambda i: (i, 0)),
    out_shape=jax.ShapeDtypeStruct((N, D_HID), jnp.float32),
)


def kernel(x, edge_index, W1, b1, W2, b2):
    deg_call, agg_call = _sc_calls()
    src = edge_index[0]
    dst = edge_index[1]
    pad_n = EROWS_PAD * CHUNK - E
    src2d = jnp.pad(src, (0, pad_n)).reshape(EROWS_PAD, CHUNK)
    dst2d = jnp.pad(dst, (0, pad_n)).reshape(EROWS_PAD, CHUNK)
    degp = deg_call(edge_index)
    dinv = lax.rsqrt(degp[:N] + degp[NPAD:NPAD + N] + 1.0)
    dv = jnp.broadcast_to(dinv[:, None], (N, H))
    g10, g11 = _tc1_call(dv, x, W1)
    t10, t11 = agg_call(src2d, dst2d, g10, g11)
    g20, g21 = _tc2_call(dv, t10, t11, g10, g11, b1.reshape(2, H), W2)
    t20, t21 = agg_call(src2d, dst2d, g20, g21)
    return _tc3_call(dv, t20, t21, g20, g21, b2.reshape(2, H))


# speedup vs baseline: 21.0428x; 1.0188x over previous
import functools

import jax
import jax.numpy as jnp
from jax import lax
from jax.experimental import pallas as pl
from jax.experimental.pallas import tpu as pltpu
from jax.experimental.pallas import tpu_sc as plsc

N = 10000
E = 160000
D_IN = 296
D_HID = 256
H = 128
NC = 2
NS = 16
NPAD = 10240
SL = NPAD // NS
CHUNK = 128
R = 2000
NBLK = N // R

EROWS = E // CHUNK
EROWS_PAD = 1256
ROWS_BIG = 80
ROWS_LAST = EROWS - (NS - 1) * ROWS_BIG
HR = ROWS_BIG // 2


def _deg_body(edge_hbm, degp_hbm, ebuf, deg_v, red_v, slots, dsem):
    c = lax.axis_index("c")
    s = lax.axis_index("s")
    w = c * NS + s
    zeros16 = jnp.zeros((16,), jnp.float32)
    ones16 = jnp.ones((16,), jnp.float32)

    def zloop(i, carry):
        deg_v[pl.ds(i * 16, 16)] = zeros16
        return carry
    lax.fori_loop(0, NPAD // 16, zloop, None)

    def hist(n, carry):
        idx = ebuf[1, pl.ds(n * 16, 16)]
        plsc.addupdate_scatter(deg_v, [idx], ones16)
        return carry

    @pl.when(w < 2)
    def _():
        pltpu.sync_copy(edge_hbm.at[:, pl.ds(w * 5120, 5120)], ebuf)
        lax.fori_loop(0, 320, hist, None)

    @pl.when(w >= 2)
    def _():
        base = 10240 + (w - 2) * 4992
        pltpu.sync_copy(edge_hbm.at[:, pl.ds(base, 4992)],
                        ebuf.at[:, pl.ds(0, 4992)])
        lax.fori_loop(0, 312, hist, None)

    pltpu.sync_copy(deg_v, slots.at[s])
    plsc.subcore_barrier()
    descs = [pltpu.async_copy(slots.at[j, pl.ds(s * SL, SL)], red_v.at[j],
                              dsem) for j in range(NS)]
    for d in descs:
        d.wait()

    def rloop(k, carry):
        acc = red_v[0, pl.ds(k * 16, 16)]
        for j in range(1, NS):
            acc = acc + red_v[j, pl.ds(k * 16, 16)]
        deg_v[pl.ds(k * 16, 16)] = acc
        return carry
    lax.fori_loop(0, SL // 16, rloop, None)
    pltpu.sync_copy(deg_v.at[pl.ds(0, SL)],
                    degp_hbm.at[pl.ds(c * NPAD + s * SL, SL)])


def _agg_body(src_hbm, dst_hbm, g0_hbm, g1_hbm, t0_hbm, t1_hbm,
              src_l, dst_l, buf_a, buf_b, zbuf, t_sh, sem_a, sem_b):
    c = lax.axis_index("c")
    s = lax.axis_index("s")
    zeros16 = jnp.zeros((16,), jnp.float32)
    base_r = s * ROWS_BIG

    for r in range(16):
        for k in range(H // 16):
            zbuf[r, pl.ds(k * 16, 16)] = zeros16

    def _zero_descs(nslabs, base):
        return [pltpu.async_copy(zbuf, t_sh.at[pl.ds(base + k * 16, 16)],
                                 sem_a) for k in range(nslabs)]

    def _load_dst(h, nload):
        pltpu.sync_copy(dst_hbm.at[pl.ds(base_r + h * HR, nload)],
                        dst_l.at[pl.ds(0, nload)])

    @pl.when(s < NS - 1)
    def _():
        descs = _zero_descs(40, s * 640)
        pltpu.sync_copy(src_hbm.at[pl.ds(base_r, ROWS_BIG)], src_l)
        _load_dst(0, HR)
        for d in descs:
            d.wait()

    @pl.when(s == NS - 1)
    def _():
        descs = _zero_descs(25, 9600)
        pltpu.sync_copy(src_hbm.at[pl.ds(base_r, 56)],
                        src_l.at[pl.ds(0, 56)])
        _load_dst(0, HR)
        for d in descs:
            d.wait()

    def _pipe(g_hbm, roff, nproc):
        def pair(i, carry):
            r0 = roff + 2 * i
            d0 = 2 * i
            desc_b = pltpu.async_copy(g_hbm.at[src_l.at[r0 + 1]], buf_b,
                                      sem_b)
            pltpu.make_async_copy(g_hbm.at[src_l.at[r0]], buf_a, sem_a).wait()
            pltpu.sync_copy(buf_a, t_sh.at[dst_l.at[d0]], add=True)

            @pl.when(i < nproc // 2 - 1)
            def _():
                pltpu.async_copy(g_hbm.at[src_l.at[r0 + 2]], buf_a, sem_a)
            desc_b.wait()
            pltpu.sync_copy(buf_b, t_sh.at[dst_l.at[d0 + 1]], add=True)
            return carry
        lax.fori_loop(0, nproc // 2, pair, None)

    def _run(g_hbm):
        pltpu.async_copy(g_hbm.at[src_l.at[0]], buf_a, sem_a)
        plsc.subcore_barrier()

        @pl.when(s < NS - 1)
        def _():
            _pipe(g_hbm, 0, HR)
            pltpu.async_copy(g_hbm.at[src_l.at[HR]], buf_a, sem_a)
            _load_dst(1, HR)
            _pipe(g_hbm, HR, HR)

        @pl.when(s == NS - 1)
        def _():
            _pipe(g_hbm, 0, HR)
            pltpu.async_copy(g_hbm.at[src_l.at[HR]], buf_a, sem_a)
            _load_dst(1, 16)
            _pipe(g_hbm, HR, ROWS_LAST - HR)

    @pl.when(c == 0)
    def _():
        _run(g0_hbm)

    @pl.when(c == 1)
    def _():
        _run(g1_hbm)

    plsc.subcore_barrier()

    def _writeback(t_hbm):
        @pl.when(s < NS - 1)
        def _():
            pltpu.sync_copy(t_sh.at[pl.ds(s * 640, 640)],
                            t_hbm.at[pl.ds(s * 640, 640)])

        @pl.when(s == NS - 1)
        def _():
            pltpu.sync_copy(t_sh.at[pl.ds(9600, 400)],
                            t_hbm.at[pl.ds(9600, 400)])

    @pl.when(c == 0)
    def _():
        _writeback(t0_hbm)

    @pl.when(c == 1)
    def _():
        _writeback(t1_hbm)


@functools.cache
def _sc_calls():
    mesh = plsc.VectorSubcoreMesh(core_axis_name="c", subcore_axis_name="s",
                                  num_cores=NC, num_subcores=NS)
    deg_call = pl.kernel(
        _deg_body,
        out_type=jax.ShapeDtypeStruct((NC * NPAD,), jnp.float32),
        mesh=mesh,
        compiler_params=pltpu.CompilerParams(needs_layout_passes=False),
        scratch_types=[
            pltpu.VMEM((2, 5120), jnp.int32),
            pltpu.VMEM((NPAD,), jnp.float32),
            pltpu.VMEM((NS, SL), jnp.float32),
            pltpu.VMEM_SHARED((NS, NPAD), jnp.float32),
            pltpu.SemaphoreType.DMA,
        ],
    )
    agg_call = pl.kernel(
        _agg_body,
        out_type=[jax.ShapeDtypeStruct((N, H), jnp.float32),
                  jax.ShapeDtypeStruct((N, H), jnp.float32)],
        mesh=mesh,
        compiler_params=pltpu.CompilerParams(needs_layout_passes=False),
        scratch_types=[
            pltpu.VMEM((ROWS_BIG, CHUNK), jnp.int32),
            pltpu.VMEM((HR, CHUNK), jnp.int32),
            pltpu.VMEM((CHUNK, H), jnp.float32),
            pltpu.VMEM((CHUNK, H), jnp.float32),
            pltpu.VMEM((16, H), jnp.float32),
            pltpu.VMEM_SHARED((N, H), jnp.float32),
            pltpu.SemaphoreType.DMA,
            pltpu.SemaphoreType.DMA,
        ],
    )
    return deg_call, agg_call


def _tc1(dv_ref, x_ref, w_ref, g0_ref, g1_ref):
    dinv = dv_ref[:, 0:1]
    xb = x_ref[...]
    nrm = jnp.sqrt(jnp.sum(xb * xb, axis=1, keepdims=True))
    hs = xb * (dinv / (nrm + 1e-8))
    g = jnp.dot(hs, w_ref[...], preferred_element_type=jnp.float32)
    g0_ref[...] = g[:, :H]
    g1_ref[...] = g[:, H:]


def _tc2(dv_ref, t0, t1, g0, g1, b1_ref, w_ref, g20_ref, g21_ref):
    dv = dv_ref[...]
    u0 = jnp.maximum(dv * (t0[...] + g0[...]) + b1_ref[0][None, :], 0.0)
    u1 = jnp.maximum(dv * (t1[...] + g1[...]) + b1_ref[1][None, :], 0.0)
    h = jnp.concatenate([u0, u1], axis=1)
    nrm = jnp.sqrt(jnp.sum(h * h, axis=1, keepdims=True))
    hs = h * (dv_ref[:, 0:1] / (nrm + 1e-8))
    g = jnp.dot(hs, w_ref[...], preferred_element_type=jnp.float32)
    g20_ref[...] = g[:, :H]
    g21_ref[...] = g[:, H:]


def _tc3(dv_ref, t0, t1, g0, g1, b2_ref, out_ref):
    dv = dv_ref[...]
    u0 = jnp.maximum(dv * (t0[...] + g0[...]) + b2_ref[0][None, :], 0.0)
    u1 = jnp.maximum(dv * (t1[...] + g1[...]) + b2_ref[1][None, :], 0.0)
    out_ref[...] = jnp.concatenate([u0, u1], axis=1)


_rspec = pl.BlockSpec((R, H), lambda i: (i, 0))

_tc1_call = pl.pallas_call(
    _tc1,
    grid=(NBLK,),
    in_specs=[
        _rspec,
        pl.BlockSpec((R, D_IN), lambda i: (i, 0)),
        pl.BlockSpec((D_IN, D_HID), lambda i: (0, 0)),
    ],
    out_specs=[_rspec, _rspec],
    out_shape=[jax.ShapeDtypeStruct((N, H), jnp.float32),
               jax.ShapeDtypeStruct((N, H), jnp.float32)],
)

_tc2_call = pl.pallas_call(
    _tc2,
    grid=(NBLK,),
    in_specs=[
        _rspec, _rspec, _rspec, _rspec, _rspec,
        pl.BlockSpec((2, H), lambda i: (0, 0)),
        pl.BlockSpec((D_HID, D_HID), lambda i: (0, 0)),
    ],
    out_specs=[_rspec, _rspec],
    out_shape=[jax.ShapeDtypeStruct((N, H), jnp.float32),
               jax.ShapeDtypeStruct((N, H), jnp.float32)],
)

_tc3_call = pl.pallas_call(
    _tc3,
    grid=(NBLK,),
    in_specs=[
        _rspec, _rspec, _rspec, _rspec, _rspec,
        pl.BlockSpec((2, H), lambda i: (0, 0)),
    ],
    out_specs=pl.BlockSpec((R, D_HID), lambda i: (i, 0)),
    out_shape=jax.ShapeDtypeStruct((N, D_HID), jnp.float32),
)


def kernel(x, edge_index, W1, b1, W2, b2):
    deg_call, agg_call = _sc_calls()
    src = edge_index[0]
    dst = edge_index[1]
    pad_n = EROWS_PAD * CHUNK - E
    src2d = jnp.pad(src, (0, pad_n)).reshape(EROWS_PAD, CHUNK)
    dst2d = jnp.pad(dst, (0, pad_n)).reshape(EROWS_PAD, CHUNK)
    degp = deg_call(edge_index)
    dinv = lax.rsqrt(degp[:N] + degp[NPAD:NPAD + N] + 1.0)
    dv = jnp.broadcast_to(dinv[:, None], (N, H))
    g10, g11 = _tc1_call(dv, x, W1)
    t10, t11 = agg_call(src2d, dst2d, g10, g11)
    g20, g21 = _tc2_call(dv, t10, t11, g10, g11, b1.reshape(2, H), W2)
    t20, t21 = agg_call(src2d, dst2d, g20, g21)
    return _tc3_call(dv, t20, t21, g20, g21, b2.reshape(2, H))


# speedup vs baseline: 21.2252x; 1.0087x over previous
import functools

import jax
import jax.numpy as jnp
from jax import lax
from jax.experimental import pallas as pl
from jax.experimental.pallas import tpu as pltpu
from jax.experimental.pallas import tpu_sc as plsc

N = 10000
E = 160000
D_IN = 296
D_HID = 256
H = 128
NC = 2
NS = 16
NPAD = 10240
SL = NPAD // NS
CHUNK = 128
R = 2000
NBLK = N // R

EROWS = E // CHUNK
EROWS_PAD = 1256
ROWS_BIG = 80
ROWS_LAST = EROWS - (NS - 1) * ROWS_BIG
HR = ROWS_BIG // 2


def _deg_body(edge_hbm, degp_hbm, ebuf, deg_v, red_v, slots, dsem):
    c = lax.axis_index("c")
    s = lax.axis_index("s")
    w = c * NS + s
    zeros16 = jnp.zeros((16,), jnp.float32)
    ones16 = jnp.ones((16,), jnp.float32)

    def zloop(i, carry):
        deg_v[pl.ds(i * 16, 16)] = zeros16
        return carry
    lax.fori_loop(0, NPAD // 16, zloop, None)

    def hist(n, carry):
        idx0 = ebuf[1, pl.ds(n * 32, 16)]
        idx1 = ebuf[1, pl.ds(n * 32 + 16, 16)]
        plsc.addupdate_scatter(deg_v, [idx0], ones16)
        plsc.addupdate_scatter(deg_v, [idx1], ones16)
        return carry

    @pl.when(w < 2)
    def _():
        pltpu.sync_copy(edge_hbm.at[:, pl.ds(w * 5120, 5120)], ebuf)
        lax.fori_loop(0, 160, hist, None)

    @pl.when(w >= 2)
    def _():
        base = 10240 + (w - 2) * 4992
        pltpu.sync_copy(edge_hbm.at[:, pl.ds(base, 4992)],
                        ebuf.at[:, pl.ds(0, 4992)])
        lax.fori_loop(0, 156, hist, None)

    pltpu.sync_copy(deg_v, slots.at[s])
    plsc.subcore_barrier()
    descs = [pltpu.async_copy(slots.at[j, pl.ds(s * SL, SL)], red_v.at[j],
                              dsem) for j in range(NS)]
    for d in descs:
        d.wait()

    def rloop(k, carry):
        acc = red_v[0, pl.ds(k * 16, 16)]
        for j in range(1, NS):
            acc = acc + red_v[j, pl.ds(k * 16, 16)]
        deg_v[pl.ds(k * 16, 16)] = acc
        return carry
    lax.fori_loop(0, SL // 16, rloop, None)
    pltpu.sync_copy(deg_v.at[pl.ds(0, SL)],
                    degp_hbm.at[pl.ds(c * NPAD + s * SL, SL)])


def _agg_body(src_hbm, dst_hbm, g0_hbm, g1_hbm, t0_hbm, t1_hbm,
              src_l, dst_l, buf_a, buf_b, zbuf, t_sh, sem_a, sem_b):
    c = lax.axis_index("c")
    s = lax.axis_index("s")
    zeros16 = jnp.zeros((16,), jnp.float32)
    base_r = s * ROWS_BIG

    for r in range(16):
        for k in range(H // 16):
            zbuf[r, pl.ds(k * 16, 16)] = zeros16

    def _zero_descs(nslabs, base):
        return [pltpu.async_copy(zbuf, t_sh.at[pl.ds(base + k * 16, 16)],
                                 sem_a) for k in range(nslabs)]

    def _load_dst(h, nload):
        pltpu.sync_copy(dst_hbm.at[pl.ds(base_r + h * HR, nload)],
                        dst_l.at[pl.ds(0, nload)])

    @pl.when(s < NS - 1)
    def _():
        descs = _zero_descs(40, s * 640)
        pltpu.sync_copy(src_hbm.at[pl.ds(base_r, ROWS_BIG)], src_l)
        _load_dst(0, HR)
        for d in descs:
            d.wait()

    @pl.when(s == NS - 1)
    def _():
        descs = _zero_descs(25, 9600)
        pltpu.sync_copy(src_hbm.at[pl.ds(base_r, 56)],
                        src_l.at[pl.ds(0, 56)])
        _load_dst(0, HR)
        for d in descs:
            d.wait()

    def _pipe(g_hbm, roff, nproc):
        def pair(i, carry):
            r0 = roff + 2 * i
            d0 = 2 * i
            desc_b = pltpu.async_copy(g_hbm.at[src_l.at[r0 + 1]], buf_b,
                                      sem_b)
            pltpu.make_async_copy(g_hbm.at[src_l.at[r0]], buf_a, sem_a).wait()
            pltpu.sync_copy(buf_a, t_sh.at[dst_l.at[d0]], add=True)

            @pl.when(i < nproc // 2 - 1)
            def _():
                pltpu.async_copy(g_hbm.at[src_l.at[r0 + 2]], buf_a, sem_a)
            desc_b.wait()
            pltpu.sync_copy(buf_b, t_sh.at[dst_l.at[d0 + 1]], add=True)
            return carry
        lax.fori_loop(0, nproc // 2, pair, None)

    def _run(g_hbm):
        pltpu.async_copy(g_hbm.at[src_l.at[0]], buf_a, sem_a)
        plsc.subcore_barrier()

        @pl.when(s < NS - 1)
        def _():
            _pipe(g_hbm, 0, HR)
            pltpu.async_copy(g_hbm.at[src_l.at[HR]], buf_a, sem_a)
            _load_dst(1, HR)
            _pipe(g_hbm, HR, HR)

        @pl.when(s == NS - 1)
        def _():
            _pipe(g_hbm, 0, HR)
            pltpu.async_copy(g_hbm.at[src_l.at[HR]], buf_a, sem_a)
            _load_dst(1, 16)
            _pipe(g_hbm, HR, ROWS_LAST - HR)

    @pl.when(c == 0)
    def _():
        _run(g0_hbm)

    @pl.when(c == 1)
    def _():
        _run(g1_hbm)

    plsc.subcore_barrier()

    def _writeback(t_hbm):
        @pl.when(s < NS - 1)
        def _():
            pltpu.sync_copy(t_sh.at[pl.ds(s * 640, 640)],
                            t_hbm.at[pl.ds(s * 640, 640)])

        @pl.when(s == NS - 1)
        def _():
            pltpu.sync_copy(t_sh.at[pl.ds(9600, 400)],
                            t_hbm.at[pl.ds(9600, 400)])

    @pl.when(c == 0)
    def _():
        _writeback(t0_hbm)

    @pl.when(c == 1)
    def _():
        _writeback(t1_hbm)


@functools.cache
def _sc_calls():
    mesh = plsc.VectorSubcoreMesh(core_axis_name="c", subcore_axis_name="s",
                                  num_cores=NC, num_subcores=NS)
    deg_call = pl.kernel(
        _deg_body,
        out_type=jax.ShapeDtypeStruct((NC * NPAD,), jnp.float32),
        mesh=mesh,
        compiler_params=pltpu.CompilerParams(needs_layout_passes=False),
        scratch_types=[
            pltpu.VMEM((2, 5120), jnp.int32),
            pltpu.VMEM((NPAD,), jnp.float32),
            pltpu.VMEM((NS, SL), jnp.float32),
            pltpu.VMEM_SHARED((NS, NPAD), jnp.float32),
            pltpu.SemaphoreType.DMA,
        ],
    )
    agg_call = pl.kernel(
        _agg_body,
        out_type=[jax.ShapeDtypeStruct((N, H), jnp.float32),
                  jax.ShapeDtypeStruct((N, H), jnp.float32)],
        mesh=mesh,
        compiler_params=pltpu.CompilerParams(needs_layout_passes=False),
        scratch_types=[
            pltpu.VMEM((ROWS_BIG, CHUNK), jnp.int32),
            pltpu.VMEM((HR, CHUNK), jnp.int32),
            pltpu.VMEM((CHUNK, H), jnp.float32),
            pltpu.VMEM((CHUNK, H), jnp.float32),
            pltpu.VMEM((16, H), jnp.float32),
            pltpu.VMEM_SHARED((N, H), jnp.float32),
            pltpu.SemaphoreType.DMA,
            pltpu.SemaphoreType.DMA,
        ],
    )
    return deg_call, agg_call


def _tc1(dv_ref, x_ref, w_ref, g0_ref, g1_ref):
    dinv = dv_ref[:, 0:1].astype(jnp.float32)
    xb = x_ref[...]
    nrm = jnp.sqrt(jnp.sum(xb * xb, axis=1, keepdims=True))
    hs = xb * (dinv / (nrm + 1e-8))
    g = jnp.dot(hs, w_ref[...], preferred_element_type=jnp.float32)
    g0_ref[...] = g[:, :H]
    g1_ref[...] = g[:, H:]


def _tc2(dv_ref, t0, t1, g0, g1, b1_ref, w_ref, g20_ref, g21_ref):
    dv = dv_ref[...].astype(jnp.float32)
    u0 = jnp.maximum(dv * (t0[...] + g0[...]) + b1_ref[0][None, :], 0.0)
    u1 = jnp.maximum(dv * (t1[...] + g1[...]) + b1_ref[1][None, :], 0.0)
    h = jnp.concatenate([u0, u1], axis=1)
    nrm = jnp.sqrt(jnp.sum(h * h, axis=1, keepdims=True))
    hs = h * (dv_ref[:, 0:1].astype(jnp.float32) / (nrm + 1e-8))
    g = jnp.dot(hs, w_ref[...], preferred_element_type=jnp.float32)
    g20_ref[...] = g[:, :H]
    g21_ref[...] = g[:, H:]


def _tc3(dv_ref, t0, t1, g0, g1, b2_ref, out_ref):
    dv = dv_ref[...].astype(jnp.float32)
    u0 = jnp.maximum(dv * (t0[...] + g0[...]) + b2_ref[0][None, :], 0.0)
    u1 = jnp.maximum(dv * (t1[...] + g1[...]) + b2_ref[1][None, :], 0.0)
    out_ref[...] = jnp.concatenate([u0, u1], axis=1)


_rspec = pl.BlockSpec((R, H), lambda i: (i, 0))

_tc1_call = pl.pallas_call(
    _tc1,
    grid=(NBLK,),
    in_specs=[
        _rspec,
        pl.BlockSpec((R, D_IN), lambda i: (i, 0)),
        pl.BlockSpec((D_IN, D_HID), lambda i: (0, 0)),
    ],
    out_specs=[_rspec, _rspec],
    out_shape=[jax.ShapeDtypeStruct((N, H), jnp.float32),
               jax.ShapeDtypeStruct((N, H), jnp.float32)],
)

_tc2_call = pl.pallas_call(
    _tc2,
    grid=(NBLK,),
    in_specs=[
        _rspec, _rspec, _rspec, _rspec, _rspec,
        pl.BlockSpec((2, H), lambda i: (0, 0)),
        pl.BlockSpec((D_HID, D_HID), lambda i: (0, 0)),
    ],
    out_specs=[_rspec, _rspec],
    out_shape=[jax.ShapeDtypeStruct((N, H), jnp.float32),
               jax.ShapeDtypeStruct((N, H), jnp.float32)],
)

_tc3_call = pl.pallas_call(
    _tc3,
    grid=(NBLK,),
    in_specs=[
        _rspec, _rspec, _rspec, _rspec, _rspec,
        pl.BlockSpec((2, H), lambda i: (0, 0)),
    ],
    out_specs=pl.BlockSpec((R, D_HID), lambda i: (i, 0)),
    out_shape=jax.ShapeDtypeStruct((N, D_HID), jnp.float32),
)


def kernel(x, edge_index, W1, b1, W2, b2):
    deg_call, agg_call = _sc_calls()
    src = edge_index[0]
    dst = edge_index[1]
    pad_n = EROWS_PAD * CHUNK - E
    src2d = jnp.pad(src, (0, pad_n)).reshape(EROWS_PAD, CHUNK)
    dst2d = jnp.pad(dst, (0, pad_n)).reshape(EROWS_PAD, CHUNK)
    degp = deg_call(edge_index)
    dinv = lax.rsqrt(degp[:N] + degp[NPAD:NPAD + N] + 1.0)
    dv = jnp.broadcast_to(dinv.astype(jnp.bfloat16)[:, None], (N, H))
    g10, g11 = _tc1_call(dv, x, W1)
    t10, t11 = agg_call(src2d, dst2d, g10, g11)
    g20, g21 = _tc2_call(dv, t10, t11, g10, g11, b1.reshape(2, H), W2)
    t20, t21 = agg_call(src2d, dst2d, g20, g21)
    return _tc3_call(dv, t20, t21, g20, g21, b2.reshape(2, H))
